# Initial kernel scaffold; baseline (speedup 1.0000x reference)
import jax
import jax.numpy as jnp
from jax.experimental import pallas as pl


def kernel(vertex, edge, context, incoming, params):
    raise NotImplementedError("write your pallas kernel here")


# speedup vs baseline: 4.9548x; 4.9548x over previous
import functools

import jax
import jax.numpy as jnp
from jax import lax
from jax.experimental import pallas as pl
from jax.experimental.pallas import tpu as pltpu
from jax.experimental.pallas import tpu_sc as plsc

B, V, D = 4, 10000, 16
N = B * V * D
BV = B * V
TROWS = B * (V + 1)
VE = EE = CE = 16
T = 2
F32 = jnp.float32

NW = 32
PER_W = N // NW
CH = 1000
NCH = PER_W // CH


def _sc_gather(table_s, table_r, idx_s, idx_r):
    mesh = plsc.VectorSubcoreMesh(core_axis_name="c", subcore_axis_name="s")

    @functools.partial(
        pl.kernel,
        mesh=mesh,
        compiler_params=pltpu.CompilerParams(use_tc_tiling_on_sc=False),
        out_type=(
            jax.ShapeDtypeStruct((N, 16), F32),
            jax.ShapeDtypeStruct((N, 32), F32),
        ),
        scratch_types=[
            pltpu.VMEM((CH,), jnp.int32),
            pltpu.VMEM((CH,), jnp.int32),
            pltpu.VMEM((CH, 16), F32),
            pltpu.VMEM((CH, 32), F32),
            pltpu.SemaphoreType.DMA,
            pltpu.SemaphoreType.DMA,
        ],
    )
    def gk(ts_hbm, tr_hbm, is_hbm, ir_hbm, sv_hbm, rw_hbm,
           is_v, ir_v, sv_v, rw_v, sem_s, sem_r):
        wid = lax.axis_index("s") * 2 + lax.axis_index("c")
        base = wid * PER_W

        def body(c, carry):
            off = base + c * CH
            pltpu.sync_copy(is_hbm.at[pl.ds(off, CH)], is_v)
            pltpu.sync_copy(ir_hbm.at[pl.ds(off, CH)], ir_v)
            a = pltpu.async_copy(ts_hbm.at[is_v], sv_v, sem_s)
            b = pltpu.async_copy(tr_hbm.at[ir_v], rw_v, sem_r)
            a.wait()
            b.wait()
            pltpu.sync_copy(sv_v, sv_hbm.at[pl.ds(off, CH)])
            pltpu.sync_copy(rw_v, rw_hbm.at[pl.ds(off, CH)])
            return carry

        lax.fori_loop(0, NCH, body, 0)

    return gk(table_s, table_r, idx_s, idx_r)


VB = 400
EB = VB * D
GRID = BV // VB
GPB = V // VB


def _relu(x):
    return jnp.maximum(x, 0.0)


def _full(w):
    return pl.BlockSpec(w.shape, lambda *g: tuple(0 for _ in w.shape))


def _enc_v_body(vx, ctx, wv1, bv1, wv2, bv2, wv3, bv3,
                wc1, bc1, wc2, bc2, wc3, bc3, out_v, out_c):
    h = _relu(vx[...] @ wv1[...] + bv1[...])
    h = _relu(h @ wv2[...] + bv2[...])
    out_v[...] = h @ wv3[...] + bv3[...]

    @pl.when(pl.program_id(0) == 0)
    def _():
        hc = _relu(ctx[...] @ wc1[...] + bc1[...])
        hc = _relu(hc @ wc2[...] + bc2[...])
        out_c[...] = hc @ wc3[...] + bc3[...]


def _enc_v(vertex_f, context, pv, pc):
    vb = 2000
    ws = (pv['W1'], pv['b1'], pv['W2'], pv['b2'], pv['W3'], pv['b3'],
          pc['W1'], pc['b1'], pc['W2'], pc['b2'], pc['W3'], pc['b3'])
    return pl.pallas_call(
        _enc_v_body,
        grid=(BV // vb,),
        in_specs=[pl.BlockSpec((vb, 8), lambda g: (g, 0)),
                  _full(context)] + [_full(w) for w in ws],
        out_specs=[pl.BlockSpec((vb, VE), lambda g: (g, 0)),
                   pl.BlockSpec((B, CE), lambda g: (0, 0))],
        out_shape=[jax.ShapeDtypeStruct((BV, VE), F32),
                   jax.ShapeDtypeStruct((B, CE), F32)],
    )(vertex_f, context, *ws)


def _mega_tail(rw, inc, embv, embc, h, wv, o_e, o_oe, o_v, o_ov, o_ge, o_gv, o_ne):
    g = pl.program_id(0)
    b = g // GPB
    mask16 = rw[:, 16:32]
    h = _relu(h @ wv['w2'][...] + wv['b2'][...])
    o80 = h @ wv['w3d'][...] + wv['bd'][...]
    e_new = o80[:, 0:16]
    o_e[...] = e_new
    dh = _relu(o80[:, 16:80])
    dh = _relu(dh @ wv['d2'][...] + wv['dd2'][...])
    lg = dh @ wv['d3'][...] + wv['dd3'][...]
    mx = jnp.max(lg, axis=1, keepdims=True)
    lse = mx + jnp.log(jnp.sum(jnp.exp(lg - mx), axis=1, keepdims=True))
    o_oe[...] = jnp.where(mask16[:, 0:2] > 0.0, lg - lse, 0.0)

    m_e = mask16 * e_new
    es = jnp.sum(m_e.reshape(VB, D, EE), axis=1)
    ns = jnp.sum((inc[...] > 0).astype(F32), axis=1, keepdims=True)
    avg_e = jnp.where(ns > 0.0, es / jnp.maximum(ns, 1.0), 0.0)
    xv = jnp.concatenate([avg_e, embv[...]], axis=1)
    ccv = embc[pl.ds(b, 1), :] @ wv['wv1c'][...]
    hv = _relu(xv @ wv['wv1'][...] + ccv + wv['bv1'][...])
    hv = _relu(hv @ wv['wv2'][...] + wv['bv2'][...])
    ov80 = hv @ wv['wv3d'][...] + wv['bvd'][...]
    v_new = ov80[:, 0:16]
    o_v[...] = v_new
    dv = _relu(ov80[:, 16:80])
    dv = _relu(dv @ wv['dv2'][...] + wv['ddv2'][...])
    o_ov[...] = dv @ wv['dv3'][...] + wv['ddv3'][...]

    @pl.when(g == 0)
    def _():
        o_ge[...] = jnp.zeros_like(o_ge)
        o_gv[...] = jnp.zeros_like(o_gv)
        o_ne[...] = jnp.zeros_like(o_ne)

    o_ge[pl.ds(b, 1), :] += jnp.sum(m_e, axis=0, keepdims=True)
    o_gv[pl.ds(b, 1), :] += jnp.sum(v_new, axis=0, keepdims=True)
    o_ne[pl.ds(b, 1), :] += jnp.broadcast_to(jnp.sum(ns), (1, 16))


_TAIL_KEYS = ('w2', 'b2', 'w3d', 'bd', 'd2', 'dd2', 'd3', 'dd3',
              'wv1', 'wv1c', 'bv1', 'wv2', 'bv2', 'wv3d', 'bvd',
              'dv2', 'ddv2', 'dv3', 'ddv3')
_T0_KEYS = ('we1', 'be1', 'we2', 'be2', 'wfa', 'wfb', 'w1c', 'b1e') + _TAIL_KEYS
_T1_KEYS = ('wfa', 'w1c', 'b1e') + _TAIL_KEYS


def _mega_body_t0(ein, sv, rw, inc, embv, embc, *rest):
    nw = len(_T0_KEYS)
    wv = dict(zip(_T0_KEYS, rest[:nw]))
    outs = rest[nw:]
    he = _relu(ein[...] @ wv['we1'][...] + wv['be1'][...])
    he = _relu(he @ wv['we2'][...] + wv['be2'][...])
    x = jnp.concatenate([rw[:, 0:16], sv[...]], axis=1)
    cc = embc[pl.ds(pl.program_id(0) // GPB, 1), :] @ wv['w1c'][...]
    h = _relu(he @ wv['wfa'][...] + x @ wv['wfb'][...] + cc + wv['b1e'][...])
    _mega_tail(rw, inc, embv, embc, h, wv, *outs)


def _mega_body_t1(ein, sv, rw, inc, embv, embc, *rest):
    nw = len(_T1_KEYS)
    wv = dict(zip(_T1_KEYS, rest[:nw]))
    outs = rest[nw:]
    x = jnp.concatenate([ein[...], rw[:, 0:16], sv[...]], axis=1)
    cc = embc[pl.ds(pl.program_id(0) // GPB, 1), :] @ wv['w1c'][...]
    h = _relu(x @ wv['wfa'][...] + cc + wv['b1e'][...])
    _mega_tail(rw, inc, embv, embc, h, wv, *outs)


def _mega(t0, ein, sv, rw, inc2d, embv, embc, wd):
    keys = _T0_KEYS if t0 else _T1_KEYS
    ein_w = ein.shape[1]
    body = _mega_body_t0 if t0 else _mega_body_t1
    return pl.pallas_call(
        body,
        grid=(GRID,),
        in_specs=[
            pl.BlockSpec((EB, ein_w), lambda g: (g, 0)),
            pl.BlockSpec((EB, 16), lambda g: (g, 0)),
            pl.BlockSpec((EB, 32), lambda g: (g, 0)),
            pl.BlockSpec((VB, D), lambda g: (g, 0)),
            pl.BlockSpec((VB, VE), lambda g: (g, 0)),
            pl.BlockSpec((B, CE), lambda g: (0, 0)),
        ] + [_full(wd[k]) for k in keys],
        out_specs=[
            pl.BlockSpec((EB, EE), lambda g: (g, 0)),
            pl.BlockSpec((EB, 2), lambda g: (g, 0)),
            pl.BlockSpec((VB, VE), lambda g: (g, 0)),
            pl.BlockSpec((VB, 2), lambda g: (g, 0)),
            pl.BlockSpec((B, EE), lambda g: (0, 0)),
            pl.BlockSpec((B, VE), lambda g: (0, 0)),
            pl.BlockSpec((B, 16), lambda g: (0, 0)),
        ],
        out_shape=[
            jax.ShapeDtypeStruct((N, EE), F32),
            jax.ShapeDtypeStruct((N, 2), F32),
            jax.ShapeDtypeStruct((BV, VE), F32),
            jax.ShapeDtypeStruct((BV, 2), F32),
            jax.ShapeDtypeStruct((B, EE), F32),
            jax.ShapeDtypeStruct((B, VE), F32),
            jax.ShapeDtypeStruct((B, 16), F32),
        ],
    )(ein, sv, rw, inc2d, embv, embc, *[wd[k] for k in keys])


def _ctx_body(ge, gv, ne, embc, w1e, w1v, w1c, b1, w2, b2, w3d, bd,
              dc2, ddc2, dc3, ddc3, o_c, o_oc):
    gee = ge[...] / ne[:, 0:1]
    gvv = gv[...] * (1.0 / V)
    h = _relu(gee @ w1e[...] + gvv @ w1v[...] + embc[...] @ w1c[...] + b1[...])
    h = _relu(h @ w2[...] + b2[...])
    o80 = h @ w3d[...] + bd[...]
    o_c[...] = o80[:, 0:16]
    dh = _relu(o80[:, 16:80])
    dh = _relu(dh @ dc2[...] + ddc2[...])
    o_oc[...] = dh @ dc3[...] + ddc3[...]


def _ctx(ge, gv, ne, embc, cw):
    args = (ge, gv, ne, embc) + cw
    return pl.pallas_call(
        _ctx_body,
        in_specs=[_full(a) for a in args],
        out_specs=[pl.BlockSpec((B, CE), lambda *g: (0, 0)),
                   pl.BlockSpec((B, 2), lambda *g: (0, 0))],
        out_shape=[jax.ShapeDtypeStruct((B, CE), F32),
                   jax.ShapeDtypeStruct((B, 2), F32)],
    )(*args)


def _prep_weights(params):
    p_e, p_v, p_c = params['e_net'], params['v_net'], params['c_net']
    d_e, d_v, d_c = params['dec_e'], params['dec_v'], params['dec_c']
    enc_e = params['enc_e']

    def row(b):
        return b.reshape(1, -1)

    wd = {}
    w1 = p_e['W1']
    wd['w1c'] = w1[48:64]
    wd['b1e'] = row(p_e['b1'])
    wd['w2'] = p_e['W2']
    wd['b2'] = row(p_e['b2'])
    wd['w3d'] = jnp.concatenate([p_e['W3'], p_e['W3'] @ d_e['W1']], axis=1)
    wd['bd'] = row(jnp.concatenate([p_e['b3'], p_e['b3'] @ d_e['W1'] + d_e['b1']]))
    wd['d2'] = d_e['W2']
    wd['dd2'] = row(d_e['b2'])
    wd['d3'] = d_e['W3']
    wd['dd3'] = row(d_e['b3'])
    wv1 = p_v['W1']
    wd['wv1'] = wv1[0:32]
    wd['wv1c'] = wv1[32:48]
    wd['bv1'] = row(p_v['b1'])
    wd['wv2'] = p_v['W2']
    wd['bv2'] = row(p_v['b2'])
    wd['wv3d'] = jnp.concatenate([p_v['W3'], p_v['W3'] @ d_v['W1']], axis=1)
    wd['bvd'] = row(jnp.concatenate([p_v['b3'], p_v['b3'] @ d_v['W1'] + d_v['b1']]))
    wd['dv2'] = d_v['W2']
    wd['ddv2'] = row(d_v['b2'])
    wd['dv3'] = d_v['W3']
    wd['ddv3'] = row(d_v['b3'])
    wd['we1'] = enc_e['W1']
    wd['be1'] = row(enc_e['b1'])
    wd['we2'] = enc_e['W2']
    wd['be2'] = row(enc_e['b2'])
    wd['wfa0'] = enc_e['W3'] @ w1[0:16]
    wd['wfb0'] = w1[16:48]
    wd['b1e0'] = row(p_e['b1'] + enc_e['b3'] @ w1[0:16])
    wd['wfa1'] = w1[0:48]
    wc1 = p_c['W1']
    cw = (wc1[0:16], wc1[16:32], wc1[32:48], row(p_c['b1']),
          p_c['W2'], row(p_c['b2']),
          jnp.concatenate([p_c['W3'], p_c['W3'] @ d_c['W1']], axis=1),
          row(jnp.concatenate([p_c['b3'], p_c['b3'] @ d_c['W1'] + d_c['b1']])),
          d_c['W2'], row(d_c['b2']), d_c['W3'], row(d_c['b3']))
    return wd, cw


def kernel(vertex, edge, context, incoming, params):
    wd, cw = _prep_weights(params)
    vertex_f = vertex.reshape(BV, 8)
    edge_f = edge.reshape(N, 4)
    inc2d = incoming.reshape(BV, D)

    boff = (jnp.arange(B, dtype=jnp.int32) * (V + 1))[:, None, None]
    idx_s = (incoming + boff).reshape(N)
    vrow = jnp.arange(V, dtype=jnp.int32)[None, :, None] + 1
    idx_r = jnp.where(incoming > 0, vrow + boff, boff).reshape(N)

    emb_v, emb_c = _enc_v(vertex_f, context, params['enc_v'], params['enc_c'])

    ones = None
    outs_v, outs_e, outs_c = [], [], []
    ein = edge_f
    for t in range(T):
        emb_v3 = emb_v.reshape(B, V, VE)
        table_s = jnp.pad(emb_v3, ((0, 0), (1, 0), (0, 0))).reshape(TROWS, 16)
        if ones is None:
            ones = jnp.ones((B, V, VE), F32)
        table_r = jnp.pad(
            jnp.concatenate([emb_v3, ones], axis=-1),
            ((0, 0), (1, 0), (0, 0))).reshape(TROWS, 32)
        sv, rw = _sc_gather(table_s, table_r, idx_s, idx_r)

        wd_t = dict(wd)
        if t == 0:
            wd_t['wfa'], wd_t['wfb'], wd_t['b1e'] = wd['wfa0'], wd['wfb0'], wd['b1e0']
        else:
            wd_t['wfa'] = wd['wfa1']
        e_new, o_e, v_new, o_v, ge, gv, ne = _mega(
            t == 0, ein, sv, rw, inc2d, emb_v, emb_c, wd_t)
        emb_c, o_c = _ctx(ge, gv, ne, emb_c, cw)
        emb_v = v_new
        ein = e_new
        outs_v.append(o_v.reshape(B, V, 2))
        outs_e.append(o_e.reshape(B, V, D, 2))
        outs_c.append(o_c)

    return (jnp.stack(outs_v), jnp.stack(outs_e), jnp.stack(outs_c))
